# baseline (speedup 1.0000x reference)
import functools

import jax
import jax.numpy as jnp
from jax import lax
from jax.experimental import pallas as pl
from jax.experimental.pallas import tpu as pltpu
from jax.experimental.pallas import tpu_sc as plsc

MARGIN = 0.2
_NC, _NS, _LANES = 2, 16, 16


def _idx_body(R, NB, dis_ref, tcol_ref, trow_ref, pidx_ref, nidx_ref):
    i = pl.program_id(0)
    n = NB * R
    dis = dis_ref[...]
    tcol = tcol_ref[...]
    trow = trow_ref[...]
    col = lax.broadcasted_iota(jnp.int32, (R, n), 1)
    row = lax.broadcasted_iota(jnp.int32, (R, n), 0) + i * R
    same = tcol == trow
    neg_inf = jnp.float32(-jnp.inf)
    pos_inf = jnp.float32(jnp.inf)
    pval = jnp.where(same & (col != row), dis, neg_inf)
    nval = jnp.where(same, pos_inf, dis)
    pmax = jnp.max(pval, axis=1, keepdims=True)
    nmin = jnp.min(nval, axis=1, keepdims=True)
    big = jnp.int32(n)
    pidx_ref[...] = jnp.min(jnp.where(pval == pmax, col, big), axis=1, keepdims=True)
    nidx_ref[...] = jnp.min(jnp.where(nval == nmin, col, big), axis=1, keepdims=True)


def _tc_indices(dis, target, R=256):
    n = target.shape[0]
    NB = n // R
    tcol = target.reshape(n, 1)
    trow = target.reshape(1, n)
    return pl.pallas_call(
        functools.partial(_idx_body, R, NB),
        grid=(NB,),
        in_specs=[
            pl.BlockSpec((R, n), lambda i: (i, 0)),
            pl.BlockSpec((R, 1), lambda i: (i, 0)),
            pl.BlockSpec((1, n), lambda i: (0, 0)),
        ],
        out_specs=[
            pl.BlockSpec((R, 1), lambda i: (i, 0)),
            pl.BlockSpec((R, 1), lambda i: (i, 0)),
        ],
        out_shape=[
            jax.ShapeDtypeStruct((n, 1), jnp.int32),
            jax.ShapeDtypeStruct((n, 1), jnp.int32),
        ],
        compiler_params=pltpu.CompilerParams(
            dimension_semantics=("arbitrary",),
        ),
    )(dis, tcol, trow)


def _make_sc_loss(n, d):
    NW = _NC * _NS
    RP = n // NW
    CH = d // _LANES
    mesh = plsc.VectorSubcoreMesh(core_axis_name="c", subcore_axis_name="s")

    @functools.partial(
        pl.kernel,
        mesh=mesh,
        out_type=jax.ShapeDtypeStruct((_NC, _LANES), jnp.float32),
        scratch_types=[
            pltpu.VMEM((RP,), jnp.int32),
            pltpu.VMEM((RP,), jnp.int32),
            pltpu.VMEM((RP, d), jnp.float32),
            pltpu.VMEM((RP, d), jnp.float32),
            pltpu.VMEM((RP, d), jnp.float32),
            pltpu.VMEM((_NS, _LANES), jnp.float32),
            pltpu.VMEM((_LANES,), jnp.float32),
            pltpu.HBM((_NC, _NS, _LANES), jnp.float32),
            pltpu.SemaphoreType.DMA,
            pltpu.SemaphoreType.DMA,
            pltpu.SemaphoreType.DMA,
        ],
        compiler_params=pltpu.CompilerParams(needs_layout_passes=False),
    )
    def sc_loss(emb_hbm, pidx_hbm, nidx_hbm, out_hbm,
                pidx_v, nidx_v, arow, prow, nrow, buf2, outbuf, stage,
                sem1, sem2, sem3):
        cid = lax.axis_index("c")
        sid = lax.axis_index("s")
        wid = sid * _NC + cid
        base = wid * RP
        pltpu.sync_copy(pidx_hbm.at[pl.ds(base, RP)], pidx_v)
        pltpu.sync_copy(nidx_hbm.at[pl.ds(base, RP)], nidx_v)
        cp1 = pltpu.async_copy(emb_hbm.at[pidx_v], prow, sem1)
        cp2 = pltpu.async_copy(emb_hbm.at[nidx_v], nrow, sem2)
        cp3 = pltpu.async_copy(emb_hbm.at[pl.ds(base, RP)], arow, sem3)
        cp1.wait()
        cp2.wait()
        cp3.wait()

        def row_body(r, total):
            acc = jnp.zeros((_LANES,), jnp.float32)
            for c in range(CH):
                a = arow[r, pl.ds(c * _LANES, _LANES)]
                p = prow[r, pl.ds(c * _LANES, _LANES)]
                ng = nrow[r, pl.ds(c * _LANES, _LANES)]
                dp = a - p
                dn = a - ng
                acc = acc + dp * dp - dn * dn
            t = jnp.sum(acc)
            return total + jnp.maximum(t + jnp.float32(MARGIN),
                                       jnp.float32(0.0))

        total = lax.fori_loop(0, RP, row_body, jnp.float32(0.0))

        lanes = lax.iota(jnp.int32, _LANES)
        zeros16 = jnp.zeros((_LANES,), jnp.float32)
        outbuf[...] = jnp.where(lanes == 0,
                                jnp.full((_LANES,), total, jnp.float32),
                                zeros16)
        pltpu.sync_copy(outbuf, stage.at[cid, sid])
        plsc.subcore_barrier()

        @pl.when(sid == 0)
        def _reduce():
            pltpu.sync_copy(stage.at[cid], buf2)
            acc = jnp.zeros((_LANES,), jnp.float32)
            for r in range(_NS):
                acc = acc + buf2[r, :]
            tot = jnp.sum(acc) * jnp.float32(1.0 / n)
            splat = jnp.full((_LANES,), tot, jnp.float32)
            outbuf[...] = jnp.where(lanes == 0, splat, zeros16)
            pltpu.sync_copy(outbuf, out_hbm.at[cid])

    return sc_loss


def kernel(embeddings, dis, target):
    n, d = embeddings.shape
    pidx2, nidx2 = _tc_indices(dis, target)
    sc_loss = _make_sc_loss(n, d)
    out = sc_loss(embeddings, pidx2.reshape(n), nidx2.reshape(n))
    return out[0, 0] + out[1, 0]
